# baseline (speedup 1.0000x reference)
import functools

import jax
import jax.numpy as jnp
from jax.experimental import pallas as pl

_K = 512


def _moc_block(x_ref, gw_ref, uw_ref, dw_ref, o_ref, *, k):
    x = x_ref[...]
    gate = jax.lax.dot_general(x, gw_ref[...], (((1,), (1,)), ((), ())),
                               preferred_element_type=jnp.float32)
    v = jax.lax.dot_general(x.astype(jnp.bfloat16),
                            uw_ref[...].astype(jnp.bfloat16),
                            (((1,), (1,)), ((), ())),
                            preferred_element_type=jnp.float32)

    bits = jax.lax.bitcast_convert_type(gate, jnp.int32)
    keys = jnp.where(bits < 0, bits ^ jnp.int32(0x7FFFFFFF), bits)

    cnt0 = jnp.sum((keys >= 0).astype(jnp.int32), axis=1)
    t = jnp.where(cnt0 >= k, jnp.int32(0), jnp.iinfo(jnp.int32).min)

    def body(i, t):
        cand = t + (jnp.int32(1) << (30 - i))
        cnt = jnp.sum((keys >= cand[:, None]).astype(jnp.int32), axis=1)
        return jnp.where(cnt >= k, cand, t)

    t = jax.lax.fori_loop(0, 31, body, t)

    act = gate * jax.nn.sigmoid(gate) * v
    act = jnp.where(keys >= t[:, None], act, 0.0)
    o_ref[...] = jax.lax.dot_general(act.astype(jnp.bfloat16),
                                     dw_ref[...].astype(jnp.bfloat16),
                                     (((1,), (1,)), ((), ())),
                                     preferred_element_type=jnp.float32)


@jax.jit
def kernel(x, gate_w, up_w, down_w):
    B, S, H = x.shape
    I = gate_w.shape[0]
    rows = B * S
    R = 256
    x2 = x.reshape(rows, H)
    out = pl.pallas_call(
        functools.partial(_moc_block, k=min(_K, I)),
        grid=(rows // R,),
        in_specs=[
            pl.BlockSpec((R, H), lambda i: (i, 0)),
            pl.BlockSpec((I, H), lambda i: (0, 0)),
            pl.BlockSpec((I, H), lambda i: (0, 0)),
            pl.BlockSpec((H, I), lambda i: (0, 0)),
        ],
        out_specs=pl.BlockSpec((R, H), lambda i: (i, 0)),
        out_shape=jax.ShapeDtypeStruct((rows, H), jnp.float32),
    )(x2, gate_w, up_w, down_w)
    return out.reshape(B, S, H)


# speedup vs baseline: 1.3533x; 1.3533x over previous
import functools

import jax
import jax.numpy as jnp
from jax.experimental import pallas as pl

_K = 512
_LOW_BIT = 8


def _moc_block(x_ref, gw_ref, uw_ref, dw_ref, o_ref, *, k):
    x = x_ref[...]
    gate = jax.lax.dot_general(gw_ref[...], x, (((1,), (1,)), ((), ())),
                               preferred_element_type=jnp.float32)
    v = jax.lax.dot_general(uw_ref[...].astype(jnp.bfloat16),
                            x.astype(jnp.bfloat16),
                            (((1,), (1,)), ((), ())),
                            preferred_element_type=jnp.float32)

    bits = jax.lax.bitcast_convert_type(gate, jnp.int32)
    keys = jnp.where(bits < 0, bits ^ jnp.int32(0x7FFFFFFF), bits)

    cnt0 = jnp.sum((keys >= 0).astype(jnp.int32), axis=0)
    t = jnp.where(cnt0 >= k, jnp.int32(0), jnp.iinfo(jnp.int32).min)

    def body(i, t):
        cand = t + (jnp.int32(1) << (30 - i))
        cnt = jnp.sum((keys >= cand[None, :]).astype(jnp.int32), axis=0)
        return jnp.where(cnt >= k, cand, t)

    t = jax.lax.fori_loop(0, 31 - _LOW_BIT, body, t)

    act = gate * jax.nn.sigmoid(gate) * v
    act = jnp.where(keys >= t[None, :], act, 0.0)
    o_ref[...] = jax.lax.dot_general(act.astype(jnp.bfloat16),
                                     dw_ref[...].astype(jnp.bfloat16),
                                     (((0,), (1,)), ((), ())),
                                     preferred_element_type=jnp.float32)


@jax.jit
def kernel(x, gate_w, up_w, down_w):
    B, S, H = x.shape
    I = gate_w.shape[0]
    rows = B * S
    R = 256
    x2 = x.reshape(rows, H)
    out = pl.pallas_call(
        functools.partial(_moc_block, k=min(_K, I)),
        grid=(rows // R,),
        in_specs=[
            pl.BlockSpec((R, H), lambda i: (i, 0)),
            pl.BlockSpec((I, H), lambda i: (0, 0)),
            pl.BlockSpec((I, H), lambda i: (0, 0)),
            pl.BlockSpec((H, I), lambda i: (0, 0)),
        ],
        out_specs=pl.BlockSpec((R, H), lambda i: (i, 0)),
        out_shape=jax.ShapeDtypeStruct((rows, H), jnp.float32),
    )(x2, gate_w, up_w, down_w)
    return out.reshape(B, S, H)
